# baseline (speedup 1.0000x reference)
import functools

import jax
import jax.numpy as jnp
from jax import lax
from jax.experimental import pallas as pl
from jax.experimental.pallas import tpu as pltpu
from jax.experimental.pallas import tpu_sc as plsc

_NQ = 8192
_K = 512
_D = 128
_CS = 2048


def _conv(x, w, stride, pad):
    return lax.conv_general_dilated(
        x, w, (stride, stride), ((pad, pad), (pad, pad)),
        dimension_numbers=('NCHW', 'OIHW', 'NCHW'))


def _conv_t(x, w, stride, pad):
    wp = jnp.flip(jnp.transpose(w, (1, 0, 2, 3)), axis=(2, 3))
    k = w.shape[2]
    p = k - 1 - pad
    return lax.conv_general_dilated(
        x, wp, (1, 1), ((p, p), (p, p)), lhs_dilation=(stride, stride),
        dimension_numbers=('NCHW', 'OIHW', 'NCHW'))


def _res_block(x, w3, w1):
    out = jax.nn.relu(x)
    out = _conv(out, w3, 1, 1)
    out = jax.nn.relu(out)
    out = _conv(out, w1, 1, 0)
    return out + x


def _vq_idx_body(zenc_ref, embd_ref, qidx_ref, tidx_ref):
    embd = embd_ref[...]
    tn = jnp.sum(embd * embd, axis=1)
    colmin = jnp.full((_K,), jnp.inf, jnp.float32)
    colarg = jnp.zeros((_K,), jnp.int32)
    for c in range(_NQ // _CS):
        q = zenc_ref[pl.ds(c * _CS, _CS), :]
        qn = jnp.sum(q * q, axis=1)
        qt = lax.dot_general(q, embd, (((1,), (1,)), ((), ())),
                             preferred_element_type=jnp.float32)
        dist = jnp.sqrt(jnp.maximum(
            qn[:, None] + tn[None, :] - 2.0 * qt, 0.0))
        rowmin = jnp.min(dist, axis=1)
        jcol = lax.broadcasted_iota(jnp.int32, (_CS, _K), 1)
        ridx = jnp.min(jnp.where(dist == rowmin[:, None], jcol, _K), axis=1)
        qidx_ref[pl.ds(c * _CS, _CS)] = ridx
        et = lax.dot_general(embd, q, (((1,), (1,)), ((), ())),
                             preferred_element_type=jnp.float32)
        dist2 = jnp.sqrt(jnp.maximum(
            tn[:, None] + qn[None, :] - 2.0 * et, 0.0))
        cmin = jnp.min(dist2, axis=1)
        icol = lax.broadcasted_iota(jnp.int32, (_K, _CS), 1)
        carg = jnp.min(jnp.where(dist2 == cmin[:, None], icol, _CS),
                       axis=1) + c * _CS
        upd = cmin < colmin
        colarg = jnp.where(upd, carg, colarg)
        colmin = jnp.where(upd, cmin, colmin)
    tidx_ref[...] = colarg


def _vq_idx(zenc, embd):
    return pl.pallas_call(
        _vq_idx_body,
        out_shape=[
            jax.ShapeDtypeStruct((_NQ,), jnp.int32),
            jax.ShapeDtypeStruct((_K,), jnp.int32),
        ],
    )(zenc, embd)


def _make_sc_gather():
    info = plsc.get_sparse_core_info()
    nw = info.num_cores * info.num_subcores
    bq = _NQ // nw
    bt = _K // nw
    mesh = plsc.VectorSubcoreMesh(core_axis_name="c", subcore_axis_name="s")

    @functools.partial(
        pl.kernel, mesh=mesh,
        out_type=[
            jax.ShapeDtypeStruct((_NQ, _D), jnp.float32),
            jax.ShapeDtypeStruct((_K, _D), jnp.float32),
        ],
        scratch_types=[
            pltpu.VMEM((bq,), jnp.int32),
            pltpu.VMEM((bq, _D), jnp.float32),
            pltpu.VMEM((bt,), jnp.int32),
            pltpu.VMEM((bt, _D), jnp.float32),
            pltpu.SemaphoreType.DMA,
        ],
    )
    def sc_gather(embd_hbm, qidx_hbm, zenc_hbm, tidx_hbm,
                  zdec_hbm, tg_hbm,
                  qidx_v, qrows_v, tidx_v, trows_v, sem):
        wid = lax.axis_index("s") * info.num_cores + lax.axis_index("c")
        qb = wid * bq
        pltpu.sync_copy(qidx_hbm.at[pl.ds(qb, bq)], qidx_v)
        pltpu.async_copy(embd_hbm.at[qidx_v], qrows_v, sem).wait()
        pltpu.sync_copy(qrows_v, zdec_hbm.at[pl.ds(qb, bq)])
        tb = wid * bt
        pltpu.sync_copy(tidx_hbm.at[pl.ds(tb, bt)], tidx_v)
        pltpu.async_copy(zenc_hbm.at[tidx_v], trows_v, sem).wait()
        pltpu.sync_copy(trows_v, tg_hbm.at[pl.ds(tb, bt)])

    return sc_gather


_sc_gather = _make_sc_gather()


def kernel(x, enc_c1, enc_c2, enc_r1w1, enc_r1w2, enc_r2w1, enc_r2w2, embd,
           dec_r1w1, dec_r1w2, dec_r2w1, dec_r2w2, dec_t1, dec_t2):
    z = _conv(x, enc_c1, 2, 1)
    z = _conv(z, enc_c2, 2, 1)
    z = _res_block(z, enc_r1w1, enc_r1w2)
    Z_enc_ori = _res_block(z, enc_r2w1, enc_r2w2)
    z_bs, z_c, z_w, z_h = Z_enc_ori.shape
    Z_enc = jnp.transpose(Z_enc_ori, (0, 2, 3, 1)).reshape(-1, _D)
    qidx, tidx = _vq_idx(Z_enc, embd)
    Z_dec_flat, Z_enc_for_embd = _sc_gather(embd, qidx, Z_enc, tidx)
    Z_dec = jnp.transpose(Z_dec_flat.reshape(z_bs, z_w, z_h, z_c), (0, 3, 1, 2))
    y = _res_block(Z_dec, dec_r1w1, dec_r1w2)
    y = _res_block(y, dec_r2w1, dec_r2w2)
    y = _conv_t(y, dec_t1, 2, 1)
    y = _conv_t(y, dec_t2, 2, 1)
    X_recon = jnp.tanh(y)
    return (X_recon, Z_enc_ori, Z_dec, Z_enc_for_embd)


# speedup vs baseline: 1.1618x; 1.1618x over previous
import jax
import jax.numpy as jnp
from jax import lax
from jax.experimental import pallas as pl
from jax.experimental.pallas import tpu as pltpu

_NQ = 8192
_K = 512
_D = 128
_CS = 2048


def _conv(x, w, stride, pad):
    return lax.conv_general_dilated(
        x, w, (stride, stride), ((pad, pad), (pad, pad)),
        dimension_numbers=('NCHW', 'OIHW', 'NCHW'))


def _conv_t(x, w, stride, pad):
    wp = jnp.flip(jnp.transpose(w, (1, 0, 2, 3)), axis=(2, 3))
    k = w.shape[2]
    p = k - 1 - pad
    return lax.conv_general_dilated(
        x, wp, (1, 1), ((p, p), (p, p)), lhs_dilation=(stride, stride),
        dimension_numbers=('NCHW', 'OIHW', 'NCHW'))


def _res_block(x, w3, w1):
    out = jax.nn.relu(x)
    out = _conv(out, w3, 1, 1)
    out = jax.nn.relu(out)
    out = _conv(out, w1, 1, 0)
    return out + x


def _split_dot(onehot, table_hi, table_lo):
    a = lax.dot_general(onehot, table_hi, (((1,), (0,)), ((), ())),
                        preferred_element_type=jnp.float32)
    b = lax.dot_general(onehot, table_lo, (((1,), (0,)), ((), ())),
                        preferred_element_type=jnp.float32)
    return a + b


def _vq_body(zenc_ref, embd_ref, zdec_ref, tgather_ref):
    embd = embd_ref[...]
    embd_hi = embd.astype(jnp.bfloat16).astype(jnp.float32)
    embd_lo = embd - embd_hi
    tn = jnp.sum(embd * embd, axis=1)
    colmin = jnp.full((_K,), jnp.inf, jnp.float32)
    colarg = jnp.zeros((_K,), jnp.int32)
    for c in range(_NQ // _CS):
        q = zenc_ref[pl.ds(c * _CS, _CS), :]
        qn = jnp.sum(q * q, axis=1)
        qt = lax.dot_general(q, embd, (((1,), (1,)), ((), ())),
                             preferred_element_type=jnp.float32)
        dist = jnp.sqrt(jnp.maximum(
            qn[:, None] + tn[None, :] - 2.0 * qt, 0.0))
        ridx = jnp.argmin(dist, axis=1).astype(jnp.int32)
        jcol = lax.broadcasted_iota(jnp.int32, (_CS, _K), 1)
        onehot = (ridx[:, None] == jcol).astype(jnp.float32)
        zdec_ref[pl.ds(c * _CS, _CS), :] = lax.dot_general(
            onehot, embd, (((1,), (0,)), ((), ())),
            preferred_element_type=jnp.float32,
            precision=lax.Precision.HIGHEST)
        et = lax.dot_general(embd, q, (((1,), (1,)), ((), ())),
                             preferred_element_type=jnp.float32)
        dist2 = jnp.sqrt(jnp.maximum(
            tn[:, None] + qn[None, :] - 2.0 * et, 0.0))
        cmin = jnp.min(dist2, axis=1)
        carg = jnp.argmin(dist2, axis=1).astype(jnp.int32) + c * _CS
        upd = cmin < colmin
        colarg = jnp.where(upd, carg, colarg)
        colmin = jnp.where(upd, cmin, colmin)
    for c in range(_NQ // _CS):
        ilocal = lax.broadcasted_iota(jnp.int32, (_K, _CS), 1) + c * _CS
        oh = (colarg[:, None] == ilocal).astype(jnp.float32)
        zc = zenc_ref[pl.ds(c * _CS, _CS), :]
        zc_hi = zc.astype(jnp.bfloat16).astype(jnp.float32)
        t = _split_dot(oh, zc_hi, zc - zc_hi)
        if c == 0:
            tgather_ref[...] = t
        else:
            tgather_ref[...] += t


def _vq(zenc, embd):
    return pl.pallas_call(
        _vq_body,
        out_shape=[
            jax.ShapeDtypeStruct((_NQ, _D), jnp.float32),
            jax.ShapeDtypeStruct((_K, _D), jnp.float32),
        ],
    )(zenc, embd)


def kernel(x, enc_c1, enc_c2, enc_r1w1, enc_r1w2, enc_r2w1, enc_r2w2, embd,
           dec_r1w1, dec_r1w2, dec_r2w1, dec_r2w2, dec_t1, dec_t2):
    z = _conv(x, enc_c1, 2, 1)
    z = _conv(z, enc_c2, 2, 1)
    z = _res_block(z, enc_r1w1, enc_r1w2)
    Z_enc_ori = _res_block(z, enc_r2w1, enc_r2w2)
    z_bs, z_c, z_w, z_h = Z_enc_ori.shape
    Z_enc = jnp.transpose(Z_enc_ori, (0, 2, 3, 1)).reshape(-1, _D)
    Z_dec_flat, Z_enc_for_embd = _vq(Z_enc, embd)
    Z_dec = jnp.transpose(Z_dec_flat.reshape(z_bs, z_w, z_h, z_c), (0, 3, 1, 2))
    y = _res_block(Z_dec, dec_r1w1, dec_r1w2)
    y = _res_block(y, dec_r2w1, dec_r2w2)
    y = _conv_t(y, dec_t1, 2, 1)
    y = _conv_t(y, dec_t2, 2, 1)
    X_recon = jnp.tanh(y)
    return (X_recon, Z_enc_ori, Z_dec, Z_enc_for_embd)
